# Initial kernel scaffold; baseline (speedup 1.0000x reference)
import jax
import jax.numpy as jnp
from jax.experimental import pallas as pl


def kernel(features, tables, gamma, beta, W1, b1, W2, b2, W3, b3):
    raise NotImplementedError("write your pallas kernel here")


# speedup vs baseline: 12.7005x; 12.7005x over previous
import functools

import jax
import jax.numpy as jnp
import numpy as np
from jax import lax
from jax.experimental import pallas as pl
from jax.experimental.pallas import tpu as pltpu
from jax.experimental.pallas import tpu_sc as plsc

B = 16384
NF = 26
VOCAB = 1000
ED = 128
IN_DIM = NF * ED
H1 = 1024
H2 = 512
EPS = 1e-5
_ISQRT = float(1.0 / np.sqrt(1.0 + EPS))

ROWS = B * NF
NC = 2
NS = 16
NW = NC * NS
ROWS_PER_W = ROWS // NW
SAMP_PER_CHUNK = 16
CHR = SAMP_PER_CHUNK * NF
CHUNKS = ROWS_PER_W // CHR


def _sc_gather(features_flat, table_flat):
    mesh = plsc.VectorSubcoreMesh(core_axis_name="c", subcore_axis_name="s")

    @functools.partial(
        pl.kernel,
        mesh=mesh,
        out_type=jax.ShapeDtypeStruct((ROWS, ED), jnp.float32),
        scratch_types=[
            pltpu.VMEM((CHR,), jnp.int32),
            pltpu.VMEM((CHR, ED), jnp.float32),
            pltpu.SemaphoreType.DMA,
        ],
    )
    def k(feat_hbm, tab_hbm, out_hbm, idx_v, rows_v, sem):
        wid = lax.axis_index("s") * NC + lax.axis_index("c")
        wbase = wid * ROWS_PER_W

        def chunk_body(c, carry):
            base = wbase + c * CHR
            pltpu.sync_copy(feat_hbm.at[pl.ds(base, CHR)], idx_v)
            for j in range(CHR // 16):
                lanes = lax.iota(jnp.int32, 16) + (j * 16)
                off = lax.rem(lanes, NF) * VOCAB
                v = idx_v[pl.ds(j * 16, 16)]
                v = jnp.minimum(jnp.maximum(v, 0), VOCAB - 1) + off
                idx_v[pl.ds(j * 16, 16)] = v
            pltpu.async_copy(tab_hbm.at[idx_v], rows_v, sem).wait()
            pltpu.sync_copy(rows_v, out_hbm.at[pl.ds(base, CHR)])
            return carry

        lax.fori_loop(0, CHUNKS, chunk_body, 0)

    return k(features_flat, table_flat)


def _mlp(x, gamma2, beta2, w1, b1r, w2, b2r, w3, b3r):
    BLK = 256
    grid = (B // BLK,)

    def body(x_ref, g_ref, be_ref, w1_ref, b1_ref, w2_ref, b2_ref,
             w3_ref, b3_ref, o_ref):
        xb = x_ref[...] * (g_ref[...] * _ISQRT) + be_ref[...]
        h = jnp.dot(xb, w1_ref[...], preferred_element_type=jnp.float32)
        h = jnp.maximum(h + b1_ref[...], 0.0)
        h = jnp.dot(h, w2_ref[...], preferred_element_type=jnp.float32)
        h = jnp.maximum(h + b2_ref[...], 0.0)
        o = jnp.dot(h, w3_ref[...], preferred_element_type=jnp.float32)
        o_ref[...] = jax.nn.sigmoid(o + b3_ref[...])

    out = pl.pallas_call(
        body,
        grid=grid,
        in_specs=[
            pl.BlockSpec((BLK, IN_DIM), lambda i: (i, 0)),
            pl.BlockSpec((1, IN_DIM), lambda i: (0, 0)),
            pl.BlockSpec((1, IN_DIM), lambda i: (0, 0)),
            pl.BlockSpec((IN_DIM, H1), lambda i: (0, 0)),
            pl.BlockSpec((1, H1), lambda i: (0, 0)),
            pl.BlockSpec((H1, H2), lambda i: (0, 0)),
            pl.BlockSpec((1, H2), lambda i: (0, 0)),
            pl.BlockSpec((H2, 1), lambda i: (0, 0)),
            pl.BlockSpec((1, 1), lambda i: (0, 0)),
        ],
        out_specs=pl.BlockSpec((BLK, 1), lambda i: (i, 0)),
        out_shape=jax.ShapeDtypeStruct((B, 1), jnp.float32),
    )(x, gamma2, beta2, w1, b1r, w2, b2r, w3, b3r)
    return out[:, 0]


def kernel(features, tables, gamma, beta, W1, b1, W2, b2, W3, b3):
    features_flat = features.astype(jnp.int32).reshape(ROWS)
    table_flat = tables.reshape(NF * VOCAB, ED)
    rows = _sc_gather(features_flat, table_flat)
    x = rows.reshape(B, IN_DIM)
    return _mlp(
        x,
        gamma.reshape(1, IN_DIM),
        beta.reshape(1, IN_DIM),
        W1,
        b1.reshape(1, H1),
        W2,
        b2.reshape(1, H2),
        W3,
        b3.reshape(1, 1),
    )


# speedup vs baseline: 17.7141x; 1.3948x over previous
import functools

import jax
import jax.numpy as jnp
import numpy as np
from jax import lax
from jax.experimental import pallas as pl
from jax.experimental.pallas import tpu as pltpu
from jax.experimental.pallas import tpu_sc as plsc

B = 16384
NF = 26
VOCAB = 1000
ED = 128
IN_DIM = NF * ED
H1 = 1024
H2 = 512
EPS = 1e-5
_ISQRT = float(1.0 / np.sqrt(1.0 + EPS))

NC = 2
NS = 16
NW = NC * NS
SAMP_PER_W = B // NW
SCH = 256
CHUNKS_PER_F = SAMP_PER_W // SCH


def _sc_gather(featT_flat, table_flat):
    mesh = plsc.VectorSubcoreMesh(core_axis_name="c", subcore_axis_name="s")
    n_iter = NF * CHUNKS_PER_F

    @functools.partial(
        pl.kernel,
        mesh=mesh,
        out_type=jax.ShapeDtypeStruct((NF, B, ED), jnp.float32),
        scratch_types=[
            pltpu.VMEM((SCH,), jnp.int32),
            pltpu.VMEM((SCH, ED), jnp.float32),
            pltpu.SemaphoreType.DMA,
        ],
    )
    def k(feat_hbm, tab_hbm, out_hbm, idx_v, rows_v, sem):
        wid = lax.axis_index("s") * NC + lax.axis_index("c")
        sbase = wid * SAMP_PER_W

        def chunk_body(i, carry):
            f = i // CHUNKS_PER_F
            s0 = sbase + (i % CHUNKS_PER_F) * SCH
            pltpu.sync_copy(feat_hbm.at[pl.ds(f * B + s0, SCH)], idx_v)
            voff = f * VOCAB
            for j in range(SCH // 16):
                v = idx_v[pl.ds(j * 16, 16)]
                v = jnp.minimum(jnp.maximum(v, 0), VOCAB - 1) + voff
                idx_v[pl.ds(j * 16, 16)] = v
            pltpu.async_copy(tab_hbm.at[idx_v], rows_v, sem).wait()
            pltpu.sync_copy(rows_v, out_hbm.at[f, pl.ds(s0, SCH)])
            return carry

        lax.fori_loop(0, n_iter, chunk_body, 0)

    return k(featT_flat, table_flat)


def _mlp(xt, gamma2, beta2, w1, b1r, w2, b2r, w3, b3r):
    BLK = 256
    grid = (B // BLK,)

    def body(xt_ref, g_ref, be_ref, w1_ref, b1_ref, w2_ref, b2_ref,
             w3_ref, b3_ref, o_ref):
        xb = jnp.concatenate([xt_ref[f] for f in range(NF)], axis=-1)
        xb = xb * (g_ref[...] * _ISQRT) + be_ref[...]
        h = jnp.dot(xb, w1_ref[...], preferred_element_type=jnp.float32)
        h = jnp.maximum(h + b1_ref[...], 0.0)
        h = jnp.dot(h, w2_ref[...], preferred_element_type=jnp.float32)
        h = jnp.maximum(h + b2_ref[...], 0.0)
        o = jnp.dot(h, w3_ref[...], preferred_element_type=jnp.float32)
        o_ref[...] = jax.nn.sigmoid(o + b3_ref[...])

    out = pl.pallas_call(
        body,
        grid=grid,
        in_specs=[
            pl.BlockSpec((NF, BLK, ED), lambda i: (0, i, 0)),
            pl.BlockSpec((1, IN_DIM), lambda i: (0, 0)),
            pl.BlockSpec((1, IN_DIM), lambda i: (0, 0)),
            pl.BlockSpec((IN_DIM, H1), lambda i: (0, 0)),
            pl.BlockSpec((1, H1), lambda i: (0, 0)),
            pl.BlockSpec((H1, H2), lambda i: (0, 0)),
            pl.BlockSpec((1, H2), lambda i: (0, 0)),
            pl.BlockSpec((H2, 1), lambda i: (0, 0)),
            pl.BlockSpec((1, 1), lambda i: (0, 0)),
        ],
        out_specs=pl.BlockSpec((BLK, 1), lambda i: (i, 0)),
        out_shape=jax.ShapeDtypeStruct((B, 1), jnp.float32),
    )(xt, gamma2, beta2, w1, b1r, w2, b2r, w3, b3r)
    return out[:, 0]


def kernel(features, tables, gamma, beta, W1, b1, W2, b2, W3, b3):
    featT_flat = features.astype(jnp.int32).T.reshape(NF * B)
    table_flat = tables.reshape(NF * VOCAB, ED)
    xt = _sc_gather(featT_flat, table_flat)
    return _mlp(
        xt,
        gamma.reshape(1, IN_DIM),
        beta.reshape(1, IN_DIM),
        W1,
        b1.reshape(1, H1),
        W2,
        b2.reshape(1, H2),
        W3,
        b3.reshape(1, 1),
    )


# speedup vs baseline: 17.7145x; 1.0000x over previous
import functools

import jax
import jax.numpy as jnp
import numpy as np
from jax import lax
from jax.experimental import pallas as pl
from jax.experimental.pallas import tpu as pltpu
from jax.experimental.pallas import tpu_sc as plsc

B = 16384
NF = 26
VOCAB = 1000
ED = 128
IN_DIM = NF * ED
H1 = 1024
H2 = 512
EPS = 1e-5
_ISQRT = float(1.0 / np.sqrt(1.0 + EPS))

NC = 2
NS = 16
NW = NC * NS
SAMP_PER_W = B // NW
SCH = 256
CHUNKS_PER_F = SAMP_PER_W // SCH


def _sc_gather(featT_flat, table_flat):
    mesh = plsc.VectorSubcoreMesh(core_axis_name="c", subcore_axis_name="s")
    n_iter = NF * CHUNKS_PER_F

    @functools.partial(
        pl.kernel,
        mesh=mesh,
        out_type=jax.ShapeDtypeStruct((NF, B, ED), jnp.float32),
        scratch_types=[
            pltpu.VMEM((SCH,), jnp.int32),
            pltpu.VMEM((SCH, ED), jnp.float32),
            pltpu.SemaphoreType.DMA,
        ],
    )
    def k(feat_hbm, tab_hbm, out_hbm, idx_v, rows_v, sem):
        wid = lax.axis_index("s") * NC + lax.axis_index("c")
        sbase = wid * SAMP_PER_W

        def chunk_body(i, carry):
            f = i // CHUNKS_PER_F
            s0 = sbase + (i % CHUNKS_PER_F) * SCH
            pltpu.sync_copy(feat_hbm.at[pl.ds(f * B + s0, SCH)], idx_v)
            voff = f * VOCAB
            for j in range(SCH // 16):
                v = idx_v[pl.ds(j * 16, 16)]
                v = jnp.minimum(jnp.maximum(v, 0), VOCAB - 1) + voff
                idx_v[pl.ds(j * 16, 16)] = v
            pltpu.async_copy(tab_hbm.at[idx_v], rows_v, sem).wait()
            pltpu.sync_copy(rows_v, out_hbm.at[f, pl.ds(s0, SCH)])
            return carry

        lax.fori_loop(0, n_iter, chunk_body, 0)

    return k(featT_flat, table_flat)


def _mlp(xt, gamma2, beta2, w1, b1r, w2, b2r, w3, b3r):
    BLK = 256
    grid = (B // BLK,)

    def body(xt_ref, g_ref, be_ref, w1_ref, b1_ref, w2_ref, b2_ref,
             w3_ref, b3_ref, o_ref):
        xb = jnp.concatenate([xt_ref[f] for f in range(NF)], axis=-1)
        xb = xb * (g_ref[...] * _ISQRT) + be_ref[...]
        h = jnp.dot(xb.astype(jnp.bfloat16), w1_ref[...].astype(jnp.bfloat16),
                    preferred_element_type=jnp.float32)
        h = jnp.maximum(h + b1_ref[...], 0.0)
        h = jnp.dot(h.astype(jnp.bfloat16), w2_ref[...].astype(jnp.bfloat16),
                    preferred_element_type=jnp.float32)
        h = jnp.maximum(h + b2_ref[...], 0.0)
        o = jnp.dot(h, w3_ref[...], preferred_element_type=jnp.float32)
        o_ref[...] = jax.nn.sigmoid(o + b3_ref[...])

    out = pl.pallas_call(
        body,
        grid=grid,
        in_specs=[
            pl.BlockSpec((NF, BLK, ED), lambda i: (0, i, 0)),
            pl.BlockSpec((1, IN_DIM), lambda i: (0, 0)),
            pl.BlockSpec((1, IN_DIM), lambda i: (0, 0)),
            pl.BlockSpec((IN_DIM, H1), lambda i: (0, 0)),
            pl.BlockSpec((1, H1), lambda i: (0, 0)),
            pl.BlockSpec((H1, H2), lambda i: (0, 0)),
            pl.BlockSpec((1, H2), lambda i: (0, 0)),
            pl.BlockSpec((H2, 1), lambda i: (0, 0)),
            pl.BlockSpec((1, 1), lambda i: (0, 0)),
        ],
        out_specs=pl.BlockSpec((BLK, 1), lambda i: (i, 0)),
        out_shape=jax.ShapeDtypeStruct((B, 1), jnp.float32),
    )(xt, gamma2, beta2, w1, b1r, w2, b2r, w3, b3r)
    return out[:, 0]


def kernel(features, tables, gamma, beta, W1, b1, W2, b2, W3, b3):
    featT_flat = features.astype(jnp.int32).T.reshape(NF * B)
    table_flat = tables.reshape(NF * VOCAB, ED)
    xt = _sc_gather(featT_flat, table_flat)
    return _mlp(
        xt,
        gamma.reshape(1, IN_DIM),
        beta.reshape(1, IN_DIM),
        W1,
        b1.reshape(1, H1),
        W2,
        b2.reshape(1, H2),
        W3,
        b3.reshape(1, 1),
    )


# speedup vs baseline: 20.1500x; 1.1375x over previous
import functools

import jax
import jax.numpy as jnp
import numpy as np
from jax import lax
from jax.experimental import pallas as pl
from jax.experimental.pallas import tpu as pltpu
from jax.experimental.pallas import tpu_sc as plsc

B = 16384
NF = 26
VOCAB = 1000
ED = 128
IN_DIM = NF * ED
H1 = 1024
H2 = 512
EPS = 1e-5
_ISQRT = float(1.0 / np.sqrt(1.0 + EPS))

NC = 2
NS = 16
NW = NC * NS
SAMP_PER_W = B // NW
SCH = 256
CHUNKS_PER_F = SAMP_PER_W // SCH


def _sc_gather(featT_flat, table_flat):
    mesh = plsc.VectorSubcoreMesh(core_axis_name="c", subcore_axis_name="s")

    @functools.partial(
        pl.kernel,
        mesh=mesh,
        out_type=jax.ShapeDtypeStruct((NF, B, ED), jnp.float32),
        scratch_types=[
            pltpu.VMEM((SCH,), jnp.int32),
            pltpu.VMEM((SCH,), jnp.int32),
            pltpu.VMEM((SCH, ED), jnp.float32),
            pltpu.VMEM((SCH, ED), jnp.float32),
            pltpu.SemaphoreType.DMA,
            pltpu.SemaphoreType.DMA,
            pltpu.SemaphoreType.DMA,
            pltpu.SemaphoreType.DMA,
        ],
    )
    def k(feat_hbm, tab_hbm, out_hbm, idx0, idx1, rows0, rows1, g0, g1, w0, w1):
        wid = lax.axis_index("s") * NC + lax.axis_index("c")
        sbase = wid * SAMP_PER_W

        def field_body(f, carry):
            base = f * B + sbase
            pltpu.sync_copy(feat_hbm.at[pl.ds(base, SCH)], idx0)
            pltpu.sync_copy(feat_hbm.at[pl.ds(base + SCH, SCH)], idx1)
            voff = f * VOCAB
            for buf in (idx0, idx1):
                for j in range(SCH // 16):
                    v = buf[pl.ds(j * 16, 16)]
                    v = jnp.minimum(jnp.maximum(v, 0), VOCAB - 1) + voff
                    buf[pl.ds(j * 16, 16)] = v

            @pl.when(f > 0)
            def _():
                pltpu.make_async_copy(
                    rows0, out_hbm.at[f - 1, pl.ds(sbase, SCH)], w0).wait()

            cp0 = pltpu.async_copy(tab_hbm.at[idx0], rows0, g0)

            @pl.when(f > 0)
            def _():
                pltpu.make_async_copy(
                    rows1, out_hbm.at[f - 1, pl.ds(sbase + SCH, SCH)], w1).wait()

            cp1 = pltpu.async_copy(tab_hbm.at[idx1], rows1, g1)
            cp0.wait()
            pltpu.async_copy(rows0, out_hbm.at[f, pl.ds(sbase, SCH)], w0)
            cp1.wait()
            pltpu.async_copy(rows1, out_hbm.at[f, pl.ds(sbase + SCH, SCH)], w1)
            return carry

        lax.fori_loop(0, NF, field_body, 0)
        pltpu.make_async_copy(
            rows0, out_hbm.at[NF - 1, pl.ds(sbase, SCH)], w0).wait()
        pltpu.make_async_copy(
            rows1, out_hbm.at[NF - 1, pl.ds(sbase + SCH, SCH)], w1).wait()

    return k(featT_flat, table_flat)


def _mlp(xt, gamma2, beta2, w1, b1r, w2, b2r, w3, b3r):
    BLK = 256
    grid = (B // BLK,)

    def body(xt_ref, g_ref, be_ref, w1_ref, b1_ref, w2_ref, b2_ref,
             w3_ref, b3_ref, o_ref):
        xb = jnp.concatenate([xt_ref[f] for f in range(NF)], axis=-1)
        xb = xb * (g_ref[...] * _ISQRT) + be_ref[...]
        h = jnp.dot(xb.astype(jnp.bfloat16), w1_ref[...].astype(jnp.bfloat16),
                    preferred_element_type=jnp.float32)
        h = jnp.maximum(h + b1_ref[...], 0.0)
        h = jnp.dot(h.astype(jnp.bfloat16), w2_ref[...].astype(jnp.bfloat16),
                    preferred_element_type=jnp.float32)
        h = jnp.maximum(h + b2_ref[...], 0.0)
        o = jnp.dot(h, w3_ref[...], preferred_element_type=jnp.float32)
        o_ref[...] = jax.nn.sigmoid(o + b3_ref[...])

    out = pl.pallas_call(
        body,
        grid=grid,
        in_specs=[
            pl.BlockSpec((NF, BLK, ED), lambda i: (0, i, 0)),
            pl.BlockSpec((1, IN_DIM), lambda i: (0, 0)),
            pl.BlockSpec((1, IN_DIM), lambda i: (0, 0)),
            pl.BlockSpec((IN_DIM, H1), lambda i: (0, 0)),
            pl.BlockSpec((1, H1), lambda i: (0, 0)),
            pl.BlockSpec((H1, H2), lambda i: (0, 0)),
            pl.BlockSpec((1, H2), lambda i: (0, 0)),
            pl.BlockSpec((H2, 1), lambda i: (0, 0)),
            pl.BlockSpec((1, 1), lambda i: (0, 0)),
        ],
        out_specs=pl.BlockSpec((BLK, 1), lambda i: (i, 0)),
        out_shape=jax.ShapeDtypeStruct((B, 1), jnp.float32),
    )(xt, gamma2, beta2, w1, b1r, w2, b2r, w3, b3r)
    return out[:, 0]


def kernel(features, tables, gamma, beta, W1, b1, W2, b2, W3, b3):
    featT_flat = features.astype(jnp.int32).T.reshape(NF * B)
    table_flat = tables.reshape(NF * VOCAB, ED)
    xt = _sc_gather(featT_flat, table_flat)
    return _mlp(
        xt,
        gamma.reshape(1, IN_DIM),
        beta.reshape(1, IN_DIM),
        W1,
        b1.reshape(1, H1),
        W2,
        b2.reshape(1, H2),
        W3,
        b3.reshape(1, 1),
    )


# speedup vs baseline: 22.8247x; 1.1327x over previous
import functools

import jax
import jax.numpy as jnp
import numpy as np
from jax import lax
from jax.experimental import pallas as pl
from jax.experimental.pallas import tpu as pltpu
from jax.experimental.pallas import tpu_sc as plsc

B = 16384
NF = 26
VOCAB = 1000
ED = 128
IN_DIM = NF * ED
H1 = 1024
H2 = 512
EPS = 1e-5
_ISQRT = float(1.0 / np.sqrt(1.0 + EPS))

NC = 2
NS = 16
NW = NC * NS
SCH = 256
N_SLICES = 2


def _sc_gather(featT_flat, table_flat, nbase, nb):
    mesh = plsc.VectorSubcoreMesh(core_axis_name="c", subcore_axis_name="s")
    nb_per_w = nb // NW
    cpf = nb_per_w // SCH
    n_ch = NF * cpf

    @functools.partial(
        pl.kernel,
        mesh=mesh,
        out_type=jax.ShapeDtypeStruct((NF, nb, ED), jnp.float32),
        scratch_types=[
            pltpu.VMEM((SCH,), jnp.int32),
            pltpu.VMEM((SCH,), jnp.int32),
            pltpu.VMEM((SCH, ED), jnp.float32),
            pltpu.VMEM((SCH, ED), jnp.float32),
            pltpu.SemaphoreType.DMA,
            pltpu.SemaphoreType.DMA,
            pltpu.SemaphoreType.DMA,
            pltpu.SemaphoreType.DMA,
        ],
    )
    def k(feat_hbm, tab_hbm, out_hbm, idx0, idx1, rows0, rows1, g0, g1, w0, w1):
        wid = lax.axis_index("s") * NC + lax.axis_index("c")
        sbase = wid * nb_per_w

        def srcoff(c):
            return (c // cpf) * B + nbase + sbase + (c % cpf) * SCH

        def dst(c):
            return out_hbm.at[c // cpf, pl.ds(sbase + (c % cpf) * SCH, SCH)]

        def wait_wb(rows, sem):
            pltpu.make_async_copy(
                rows, out_hbm.at[0, pl.ds(sbase, SCH)], sem).wait()

        def pair_body(p, carry):
            c0 = 2 * p
            c1 = 2 * p + 1
            pltpu.sync_copy(feat_hbm.at[pl.ds(srcoff(c0), SCH)], idx0)
            pltpu.sync_copy(feat_hbm.at[pl.ds(srcoff(c1), SCH)], idx1)
            for buf, c in ((idx0, c0), (idx1, c1)):
                voff = (c // cpf) * VOCAB
                for j in range(SCH // 16):
                    v = buf[pl.ds(j * 16, 16)]
                    v = jnp.minimum(jnp.maximum(v, 0), VOCAB - 1) + voff
                    buf[pl.ds(j * 16, 16)] = v

            @pl.when(p > 0)
            def _():
                wait_wb(rows0, w0)

            cp0 = pltpu.async_copy(tab_hbm.at[idx0], rows0, g0)

            @pl.when(p > 0)
            def _():
                wait_wb(rows1, w1)

            cp1 = pltpu.async_copy(tab_hbm.at[idx1], rows1, g1)
            cp0.wait()
            pltpu.async_copy(rows0, dst(c0), w0)
            cp1.wait()
            pltpu.async_copy(rows1, dst(c1), w1)
            return carry

        lax.fori_loop(0, n_ch // 2, pair_body, 0)
        wait_wb(rows0, w0)
        wait_wb(rows1, w1)

    return k(featT_flat, table_flat)


def _mlp(xt, gamma2, beta2, w1, b1r, w2, b2r, w3, b3r, nb):
    BLK = 256
    grid = (nb // BLK,)

    def body(xt_ref, g_ref, be_ref, w1_ref, b1_ref, w2_ref, b2_ref,
             w3_ref, b3_ref, o_ref):
        xb = jnp.concatenate([xt_ref[f] for f in range(NF)], axis=-1)
        xb = xb * (g_ref[...] * _ISQRT) + be_ref[...]
        h = jnp.dot(xb.astype(jnp.bfloat16), w1_ref[...].astype(jnp.bfloat16),
                    preferred_element_type=jnp.float32)
        h = jnp.maximum(h + b1_ref[...], 0.0)
        h = jnp.dot(h.astype(jnp.bfloat16), w2_ref[...].astype(jnp.bfloat16),
                    preferred_element_type=jnp.float32)
        h = jnp.maximum(h + b2_ref[...], 0.0)
        o = jnp.dot(h, w3_ref[...], preferred_element_type=jnp.float32)
        o_ref[...] = jax.nn.sigmoid(o + b3_ref[...])

    out = pl.pallas_call(
        body,
        grid=grid,
        in_specs=[
            pl.BlockSpec((NF, BLK, ED), lambda i: (0, i, 0)),
            pl.BlockSpec((1, IN_DIM), lambda i: (0, 0)),
            pl.BlockSpec((1, IN_DIM), lambda i: (0, 0)),
            pl.BlockSpec((IN_DIM, H1), lambda i: (0, 0)),
            pl.BlockSpec((1, H1), lambda i: (0, 0)),
            pl.BlockSpec((H1, H2), lambda i: (0, 0)),
            pl.BlockSpec((1, H2), lambda i: (0, 0)),
            pl.BlockSpec((H2, 1), lambda i: (0, 0)),
            pl.BlockSpec((1, 1), lambda i: (0, 0)),
        ],
        out_specs=pl.BlockSpec((BLK, 1), lambda i: (i, 0)),
        out_shape=jax.ShapeDtypeStruct((nb, 1), jnp.float32),
    )(xt, gamma2, beta2, w1, b1r, w2, b2r, w3, b3r)
    return out[:, 0]


def kernel(features, tables, gamma, beta, W1, b1, W2, b2, W3, b3):
    featT_flat = features.astype(jnp.int32).T.reshape(NF * B)
    table_flat = tables.reshape(NF * VOCAB, ED)
    g2 = gamma.reshape(1, IN_DIM)
    be2 = beta.reshape(1, IN_DIM)
    b1r = b1.reshape(1, H1)
    b2r = b2.reshape(1, H2)
    b3r = b3.reshape(1, 1)
    nb = B // N_SLICES
    outs = []
    for s in range(N_SLICES):
        xt = _sc_gather(featT_flat, table_flat, s * nb, nb)
        outs.append(_mlp(xt, g2, be2, W1, b1r, W2, b2r, W3, b3r, nb))
    return jnp.concatenate(outs)


# speedup vs baseline: 23.4937x; 1.0293x over previous
import functools

import jax
import jax.numpy as jnp
import numpy as np
from jax import lax
from jax.experimental import pallas as pl
from jax.experimental.pallas import tpu as pltpu
from jax.experimental.pallas import tpu_sc as plsc

B = 16384
NF = 26
VOCAB = 1000
ED = 128
IN_DIM = NF * ED
H1 = 1024
H2 = 512
EPS = 1e-5
_ISQRT = float(1.0 / np.sqrt(1.0 + EPS))

NC = 2
NS = 16
NW = NC * NS
N_SLICES = 4


def _sc_gather(featT_flat, table_flat, nbase, nb):
    mesh = plsc.VectorSubcoreMesh(core_axis_name="c", subcore_axis_name="s")
    nb_per_w = nb // NW
    SCH = min(256, nb_per_w)
    cpf = nb_per_w // SCH
    n_ch = NF * cpf

    @functools.partial(
        pl.kernel,
        mesh=mesh,
        out_type=jax.ShapeDtypeStruct((NF, nb, ED), jnp.float32),
        scratch_types=[
            pltpu.VMEM((SCH,), jnp.int32),
            pltpu.VMEM((SCH,), jnp.int32),
            pltpu.VMEM((SCH, ED), jnp.float32),
            pltpu.VMEM((SCH, ED), jnp.float32),
            pltpu.SemaphoreType.DMA,
            pltpu.SemaphoreType.DMA,
            pltpu.SemaphoreType.DMA,
            pltpu.SemaphoreType.DMA,
        ],
    )
    def k(feat_hbm, tab_hbm, out_hbm, idx0, idx1, rows0, rows1, g0, g1, w0, w1):
        wid = lax.axis_index("s") * NC + lax.axis_index("c")
        sbase = wid * nb_per_w

        def srcoff(c):
            return (c // cpf) * B + nbase + sbase + (c % cpf) * SCH

        def dst(c):
            return out_hbm.at[c // cpf, pl.ds(sbase + (c % cpf) * SCH, SCH)]

        def wait_wb(rows, sem):
            pltpu.make_async_copy(
                rows, out_hbm.at[0, pl.ds(sbase, SCH)], sem).wait()

        def pair_body(p, carry):
            c0 = 2 * p
            c1 = 2 * p + 1
            pltpu.sync_copy(feat_hbm.at[pl.ds(srcoff(c0), SCH)], idx0)
            pltpu.sync_copy(feat_hbm.at[pl.ds(srcoff(c1), SCH)], idx1)
            for buf, c in ((idx0, c0), (idx1, c1)):
                voff = (c // cpf) * VOCAB
                for j in range(SCH // 16):
                    v = buf[pl.ds(j * 16, 16)]
                    v = jnp.minimum(jnp.maximum(v, 0), VOCAB - 1) + voff
                    buf[pl.ds(j * 16, 16)] = v

            @pl.when(p > 0)
            def _():
                wait_wb(rows0, w0)

            cp0 = pltpu.async_copy(tab_hbm.at[idx0], rows0, g0)

            @pl.when(p > 0)
            def _():
                wait_wb(rows1, w1)

            cp1 = pltpu.async_copy(tab_hbm.at[idx1], rows1, g1)
            cp0.wait()
            pltpu.async_copy(rows0, dst(c0), w0)
            cp1.wait()
            pltpu.async_copy(rows1, dst(c1), w1)
            return carry

        lax.fori_loop(0, n_ch // 2, pair_body, 0)
        wait_wb(rows0, w0)
        wait_wb(rows1, w1)

    return k(featT_flat, table_flat)


def _mlp(xt, gamma2, beta2, w1, b1r, w2, b2r, w3, b3r, nb):
    BLK = 256
    grid = (nb // BLK,)

    def body(xt_ref, g_ref, be_ref, w1_ref, b1_ref, w2_ref, b2_ref,
             w3_ref, b3_ref, o_ref):
        xb = jnp.concatenate([xt_ref[f] for f in range(NF)], axis=-1)
        xb = xb * (g_ref[...] * _ISQRT) + be_ref[...]
        h = jnp.dot(xb.astype(jnp.bfloat16), w1_ref[...].astype(jnp.bfloat16),
                    preferred_element_type=jnp.float32)
        h = jnp.maximum(h + b1_ref[...], 0.0)
        h = jnp.dot(h.astype(jnp.bfloat16), w2_ref[...].astype(jnp.bfloat16),
                    preferred_element_type=jnp.float32)
        h = jnp.maximum(h + b2_ref[...], 0.0)
        o = jnp.dot(h, w3_ref[...], preferred_element_type=jnp.float32)
        o_ref[...] = jax.nn.sigmoid(o + b3_ref[...])

    out = pl.pallas_call(
        body,
        grid=grid,
        in_specs=[
            pl.BlockSpec((NF, BLK, ED), lambda i: (0, i, 0)),
            pl.BlockSpec((1, IN_DIM), lambda i: (0, 0)),
            pl.BlockSpec((1, IN_DIM), lambda i: (0, 0)),
            pl.BlockSpec((IN_DIM, H1), lambda i: (0, 0)),
            pl.BlockSpec((1, H1), lambda i: (0, 0)),
            pl.BlockSpec((H1, H2), lambda i: (0, 0)),
            pl.BlockSpec((1, H2), lambda i: (0, 0)),
            pl.BlockSpec((H2, 1), lambda i: (0, 0)),
            pl.BlockSpec((1, 1), lambda i: (0, 0)),
        ],
        out_specs=pl.BlockSpec((BLK, 1), lambda i: (i, 0)),
        out_shape=jax.ShapeDtypeStruct((nb, 1), jnp.float32),
    )(xt, gamma2, beta2, w1, b1r, w2, b2r, w3, b3r)
    return out[:, 0]


def kernel(features, tables, gamma, beta, W1, b1, W2, b2, W3, b3):
    featT_flat = features.astype(jnp.int32).T.reshape(NF * B)
    table_flat = tables.reshape(NF * VOCAB, ED)
    g2 = gamma.reshape(1, IN_DIM)
    be2 = beta.reshape(1, IN_DIM)
    b1r = b1.reshape(1, H1)
    b2r = b2.reshape(1, H2)
    b3r = b3.reshape(1, 1)
    nb = B // N_SLICES
    outs = []
    for s in range(N_SLICES):
        xt = _sc_gather(featT_flat, table_flat, s * nb, nb)
        outs.append(_mlp(xt, g2, be2, W1, b1r, W2, b2r, W3, b3r, nb))
    return jnp.concatenate(outs)
